# Initial kernel scaffold; baseline (speedup 1.0000x reference)
import jax
import jax.numpy as jnp
from jax.experimental import pallas as pl


def kernel(X, emb_table, counts):
    raise NotImplementedError("write your pallas kernel here")


# speedup vs baseline: 8.9727x; 8.9727x over previous
import functools

import jax
import jax.numpy as jnp
from jax import lax
from jax.experimental import pallas as pl
from jax.experimental.pallas import tpu as pltpu
from jax.experimental.pallas import tpu_sc as plsc

_F = 26
_V = 20
_E = 64
_A = 20.0


def _tc_prep(x_ref, prim_ref, dflt_ref, cnt_ref, blend_ref, fidx_ref):
    c = cnt_ref[...].astype(jnp.float32)
    w = (c / (c + _A))[:, None]
    blend_ref[...] = w * prim_ref[...] + (1.0 - w) * dflt_ref[...]
    fofs = lax.broadcasted_iota(jnp.int32, x_ref.shape, 1) * _V
    fidx_ref[...] = x_ref[...] + fofs


def kernel(X, emb_table, counts):
    B = X.shape[0]
    R = B * _F

    emb3 = emb_table.reshape(_F, _V + 1, _E)
    prim = emb3[:, 1:, :].reshape(_F * _V, _E)
    dflt = jnp.broadcast_to(emb3[:, :1, :], (_F, _V, _E)).reshape(_F * _V, _E)
    cnt_flat = counts.reshape(_F * _V)

    blend, fidx = pl.pallas_call(
        _tc_prep,
        out_shape=(
            jax.ShapeDtypeStruct((_F * _V, _E), jnp.float32),
            jax.ShapeDtypeStruct((B, _F), jnp.int32),
        ),
    )(X, prim, dflt, cnt_flat)

    info = plsc.get_sparse_core_info()
    NC, NS = info.num_cores, info.num_subcores
    NW = NC * NS
    RPW = R // NW
    CH = 128
    NCH = RPW // CH
    assert RPW % CH == 0 and R % NW == 0

    fidx3 = fidx.reshape(NW, NCH, CH)

    mesh = plsc.VectorSubcoreMesh(core_axis_name="c", subcore_axis_name="s")

    @functools.partial(
        pl.kernel,
        out_type=jax.ShapeDtypeStruct((R, _E), jnp.float32),
        mesh=mesh,
        compiler_params=pltpu.CompilerParams(use_tc_tiling_on_sc=False),
        scratch_types=[
            pltpu.VMEM((NCH, CH), jnp.int32),
            pltpu.VMEM((CH, _E), jnp.float32),
            pltpu.SemaphoreType.DMA,
        ],
    )
    def sc_gather(fidx_hbm, blend_hbm, out_hbm, idx_v, buf, gsem):
        wid = lax.axis_index("s") * NC + lax.axis_index("c")
        base = wid * RPW
        pltpu.sync_copy(fidx_hbm.at[wid], idx_v)

        def body(j, carry):
            pltpu.async_copy(blend_hbm.at[idx_v.at[j]], buf, gsem).wait()
            pltpu.sync_copy(buf, out_hbm.at[pl.ds(base + j * CH, CH)])
            return carry

        lax.fori_loop(0, NCH, body, 0)

    out = sc_gather(fidx3, blend)
    return out.reshape(B, _F, _E)


# speedup vs baseline: 9.1475x; 1.0195x over previous
import functools

import jax
import jax.numpy as jnp
from jax import lax
from jax.experimental import pallas as pl
from jax.experimental.pallas import tpu as pltpu
from jax.experimental.pallas import tpu_sc as plsc

_F = 26
_V = 20
_E = 64
_A = 20.0


def _tc_prep(x_ref, prim_ref, dflt_ref, cnt_ref, blend_ref, fidx_ref):
    c = cnt_ref[...].astype(jnp.float32)
    w = (c / (c + _A))[:, None]
    blend_ref[...] = w * prim_ref[...] + (1.0 - w) * dflt_ref[...]
    fofs = lax.broadcasted_iota(jnp.int32, x_ref.shape, 1) * _V
    fidx_ref[...] = x_ref[...] + fofs


def kernel(X, emb_table, counts):
    B = X.shape[0]
    R = B * _F

    emb3 = emb_table.reshape(_F, _V + 1, _E)
    prim = emb3[:, 1:, :].reshape(_F * _V, _E)
    dflt = jnp.broadcast_to(emb3[:, :1, :], (_F, _V, _E)).reshape(_F * _V, _E)
    cnt_flat = counts.reshape(_F * _V)

    blend, fidx = pl.pallas_call(
        _tc_prep,
        out_shape=(
            jax.ShapeDtypeStruct((_F * _V, _E), jnp.float32),
            jax.ShapeDtypeStruct((B, _F), jnp.int32),
        ),
    )(X, prim, dflt, cnt_flat)

    info = plsc.get_sparse_core_info()
    NC, NS = info.num_cores, info.num_subcores
    NW = NC * NS
    RPW = R // NW
    CH = 128
    NCH = RPW // CH
    assert RPW % CH == 0 and R % NW == 0

    fidx3 = fidx.reshape(NW, NCH, CH)

    mesh = plsc.VectorSubcoreMesh(core_axis_name="c", subcore_axis_name="s")

    @functools.partial(
        pl.kernel,
        out_type=jax.ShapeDtypeStruct((R, _E), jnp.float32),
        mesh=mesh,
        compiler_params=pltpu.CompilerParams(use_tc_tiling_on_sc=False),
        scratch_types=[
            pltpu.VMEM((NCH, CH), jnp.int32),
            pltpu.VMEM((CH, _E), jnp.float32),
            pltpu.VMEM((CH, _E), jnp.float32),
            pltpu.SemaphoreType.DMA,
            pltpu.SemaphoreType.DMA,
        ],
    )
    def sc_gather(fidx_hbm, blend_hbm, out_hbm, idx_v, buf0, buf1, gsem0, gsem1):
        wid = lax.axis_index("s") * NC + lax.axis_index("c")
        base = wid * RPW
        pltpu.sync_copy(fidx_hbm.at[wid], idx_v)
        pltpu.async_copy(blend_hbm.at[idx_v.at[0]], buf0, gsem0)

        def body(i, carry):
            j0 = 2 * i
            pltpu.async_copy(blend_hbm.at[idx_v.at[j0 + 1]], buf1, gsem1)
            pltpu.make_async_copy(blend_hbm.at[idx_v.at[j0]], buf0, gsem0).wait()
            pltpu.sync_copy(buf0, out_hbm.at[pl.ds(base + j0 * CH, CH)])

            @pl.when(i < NCH // 2 - 1)
            def _():
                pltpu.async_copy(blend_hbm.at[idx_v.at[j0 + 2]], buf0, gsem0)

            pltpu.make_async_copy(blend_hbm.at[idx_v.at[j0 + 1]], buf1, gsem1).wait()
            pltpu.sync_copy(buf1, out_hbm.at[pl.ds(base + (j0 + 1) * CH, CH)])
            return carry

        lax.fori_loop(0, NCH // 2, body, 0)

    out = sc_gather(fidx3, blend)
    return out.reshape(B, _F, _E)


# speedup vs baseline: 13.5812x; 1.4847x over previous
import functools

import jax
import jax.numpy as jnp
from jax import lax
from jax.experimental import pallas as pl
from jax.experimental.pallas import tpu as pltpu
from jax.experimental.pallas import tpu_sc as plsc

_F = 26
_V = 20
_E = 64
_A = 20.0
_NV = _F * _V
_NVP = 528


def _tc_prep(xt_ref, primt_ref, dfltt_ref, cnt_ref, blendt_ref, fidxt_ref):
    c = cnt_ref[...].astype(jnp.float32)
    w = (c / (c + _A))[None, :]
    blendt_ref[...] = w * primt_ref[...] + (1.0 - w) * dfltt_ref[...]
    fofs = lax.broadcasted_iota(jnp.int32, xt_ref.shape, 0) * _V
    fidxt_ref[...] = xt_ref[...] + fofs


def kernel(X, emb_table, counts):
    B = X.shape[0]
    NBT = B // 128

    emb3 = emb_table.reshape(_F, _V + 1, _E)
    primt = jnp.transpose(emb3[:, 1:, :], (2, 0, 1)).reshape(_E, _NV)
    dfltt = jnp.repeat(jnp.transpose(emb3[:, 0, :], (1, 0)), _V, axis=1)
    primt = jnp.pad(primt, ((0, 0), (0, _NVP - _NV)))
    dfltt = jnp.pad(dfltt, ((0, 0), (0, _NVP - _NV)))
    cntp = jnp.pad(counts.reshape(_NV), (0, _NVP - _NV))
    XT = jnp.transpose(X, (1, 0))

    blendt, fidxt = pl.pallas_call(
        _tc_prep,
        out_shape=(
            jax.ShapeDtypeStruct((_E, _NVP), jnp.float32),
            jax.ShapeDtypeStruct((_F, B), jnp.int32),
        ),
    )(XT, primt, dfltt, cntp)

    info = plsc.get_sparse_core_info()
    NC, NS = info.num_cores, info.num_subcores
    NW = NC * NS
    NCHUNK = _F * NBT
    CPW = NCHUNK // NW
    fidx2 = fidxt.reshape(NCHUNK, 128)

    mesh = plsc.VectorSubcoreMesh(core_axis_name="c", subcore_axis_name="s")

    @functools.partial(
        pl.kernel,
        out_type=jax.ShapeDtypeStruct((_F, 8, NBT, 8, 128), jnp.float32),
        mesh=mesh,
        compiler_params=pltpu.CompilerParams(
            use_tc_tiling_on_sc=False, needs_layout_passes=False
        ),
        scratch_types=[
            pltpu.VMEM((_E, _NVP), jnp.float32),
            pltpu.VMEM((CPW, 128), jnp.int32),
            pltpu.VMEM((8, 8, 128), jnp.float32),
        ],
    )
    def sc_fill(fidx_hbm, blendt_hbm, out_hbm, tbl_v, idx_v, obuf):
        wid = lax.axis_index("s") * NC + lax.axis_index("c")
        pltpu.sync_copy(blendt_hbm, tbl_v)
        pltpu.sync_copy(fidx_hbm.at[pl.ds(wid * CPW, CPW)], idx_v)

        def chunk(j, carry):
            t = wid * CPW + j
            f = t // NBT
            bt = t % NBT
            for c in range(8):
                idx16 = idx_v[j, pl.ds(c * 16, 16)]
                for e in range(_E):
                    vals = plsc.load_gather(
                        tbl_v, [jnp.full((16,), e, jnp.int32), idx16]
                    )
                    obuf[e // 8, e % 8, pl.ds(c * 16, 16)] = vals
            pltpu.sync_copy(obuf, out_hbm.at[f, :, bt])
            return carry

        lax.fori_loop(0, CPW, chunk, 0)

    q = sc_fill(fidx2, blendt)
    return q.transpose((2, 4, 0, 1, 3)).reshape(B, _F, _E)
